# baseline (speedup 1.0000x reference)
import functools

import jax
import jax.numpy as jnp
from jax import lax
from jax.experimental import pallas as pl
from jax.experimental.pallas import tpu as pltpu
from jax.experimental.pallas import tpu_sc as plsc

NC = 2
NS = 16


def _spmm(h, src2d, dst2d, feat_split):
    if feat_split:
        _, N, W = h.shape
    else:
        N, W = h.shape
    R = src2d.shape[0]
    CH = 128
    TPW = NS if feat_split else NC * NS
    RPTile = R // TPW
    XTRA = R - TPW * RPTile
    K = 6
    NJ = RPTile // K
    assert RPTile == NJ * K and NJ >= 2
    RPT = (N // NS) & ~7
    REM = N - NS * RPT
    nzc = RPT // CH
    zrem = RPT - nzc * CH

    mesh = plsc.VectorSubcoreMesh(core_axis_name="c", subcore_axis_name="s",
                                  num_cores=NC, num_subcores=NS)

    def body(h_hbm, s_hbm, d_hbm, z_hbm, out_hbm, acc, srcA, dstA,
             r0b, r1b, r2b, r3b, r4b, r5b, xsrc, xdst,
             g0, g1, g2, g3, g4, g5, s0, s1, s2, s3, s4, s5):
        c = lax.axis_index("c")
        s = lax.axis_index("s")
        rows = [r0b, r1b, r2b, r3b, r4b, r5b]
        gsem = [g0, g1, g2, g3, g4, g5]
        ssem = [s0, s1, s2, s3, s4, s5]
        table = h_hbm.at[c] if feat_split else h_hbm

        pltpu.sync_copy(z_hbm, rows[0])
        r0 = s * RPT
        for k in range(nzc):
            pltpu.sync_copy(rows[0], acc.at[pl.ds(r0 + k * CH, CH)])
        if zrem:
            pltpu.sync_copy(rows[0].at[pl.ds(0, zrem)],
                            acc.at[pl.ds(r0 + nzc * CH, zrem)])
        if REM:
            @pl.when(s == 0)
            def _():
                pltpu.sync_copy(rows[0].at[pl.ds(0, REM)],
                                acc.at[pl.ds(NS * RPT, REM)])

        rowbase = (s if feat_split else c * NS + s) * RPTile
        pltpu.sync_copy(s_hbm.at[pl.ds(rowbase, RPTile)], srcA)
        pltpu.sync_copy(d_hbm.at[pl.ds(rowbase, RPTile)], dstA)
        plsc.subcore_barrier()

        def gstart(t, v):
            pltpu.async_copy(table.at[srcA.at[t]], rows[v], gsem[v])

        def gwait(t, v):
            pltpu.make_async_copy(table.at[srcA.at[t]], rows[v],
                                  gsem[v]).wait()

        def sstart(t, v):
            pltpu.async_copy(rows[v], acc.at[dstA.at[t]], ssem[v], add=True)

        def swait(t, v):
            pltpu.make_async_copy(rows[v], acc.at[dstA.at[t]],
                                  ssem[v]).wait()

        for v in range(K):
            gstart(v, v)
            if v >= 1:
                gwait(v - 1, v - 1)
                sstart(v - 1, v - 1)

        def rev(j, _):
            for v in range(K):
                t = K * j + v
                pv = (v - 1) % K
                swait(t - K, v)
                gstart(t, v)
                gwait(t - 1, pv)
                sstart(t - 1, pv)
            return 0
        lax.fori_loop(1, NJ, rev, 0)

        last = RPTile - 1
        gwait(last, K - 1)
        sstart(last, K - 1)
        for v in range(K):
            swait(RPTile - K + v, v)

        if XTRA:
            u = s if feat_split else c * NS + s

            @pl.when(u < XTRA)
            def _():
                xr = TPW * RPTile + u
                pltpu.sync_copy(s_hbm.at[xr], xsrc)
                pltpu.sync_copy(d_hbm.at[xr], xdst)
                pltpu.async_copy(table.at[xsrc], rows[0], gsem[0]).wait()
                pltpu.sync_copy(rows[0], acc.at[xdst], add=True)
        plsc.subcore_barrier()

        pltpu.sync_copy(acc.at[pl.ds(r0, RPT)],
                        out_hbm.at[c, pl.ds(r0, RPT)])
        if REM:
            @pl.when(s == 0)
            def _():
                pltpu.sync_copy(acc.at[pl.ds(NS * RPT, REM)],
                                out_hbm.at[c, pl.ds(NS * RPT, REM)])

    f = pl.kernel(
        body,
        out_type=jax.ShapeDtypeStruct((NC, N, W), jnp.float32),
        mesh=mesh,
        scratch_types=(
            [pltpu.VMEM_SHARED((N, W), jnp.float32),
             pltpu.VMEM((RPTile, CH), jnp.int32),
             pltpu.VMEM((RPTile, CH), jnp.int32)]
            + [pltpu.VMEM((CH, W), jnp.float32) for _ in range(K)]
            + [pltpu.VMEM((CH,), jnp.int32),
               pltpu.VMEM((CH,), jnp.int32)]
            + [pltpu.SemaphoreType.DMA for _ in range(2 * K)]
        ),
        compiler_params=pltpu.CompilerParams(use_tc_tiling_on_sc=False),
    )
    return f(h, src2d, dst2d, jnp.zeros((CH, W), jnp.float32))


_BLK = 1000


def _mm_act(x, w, b):
    Nn, F = x.shape
    H = w.shape[1]
    Hh = H // 2

    def kern(x_ref, w_ref, b_ref, o_ref):
        v = jnp.tanh(
            jnp.dot(x_ref[...], w_ref[...],
                    preferred_element_type=jnp.float32) + b_ref[...])
        o_ref[0] = v[:, :Hh]
        o_ref[1] = v[:, Hh:]

    return pl.pallas_call(
        kern,
        grid=(Nn // _BLK,),
        in_specs=[pl.BlockSpec((_BLK, F), lambda i: (i, 0)),
                  pl.BlockSpec((F, H), lambda i: (0, 0)),
                  pl.BlockSpec((1, H), lambda i: (0, 0))],
        out_specs=pl.BlockSpec((2, _BLK, Hh), lambda i: (0, i, 0)),
        out_shape=jax.ShapeDtypeStruct((2, Nn, Hh), jnp.float32),
    )(x, w, b.reshape(1, -1))


def _gatelin(h, gW, gb, lW, lb):
    _, Nn, Hh = h.shape
    H = 2 * Hh

    def kern(h_ref, gw, gb_, lw, lb_, g_ref, l_ref):
        hh = jnp.concatenate([h_ref[0], h_ref[1]], axis=1)
        g_ref[...] = jax.nn.sigmoid(
            jnp.dot(hh, gw[...], preferred_element_type=jnp.float32)
            + gb_[...])
        l_ref[...] = (jnp.dot(hh, lw[...],
                              preferred_element_type=jnp.float32) + lb_[...])

    return pl.pallas_call(
        kern,
        grid=(Nn // _BLK,),
        in_specs=[pl.BlockSpec((2, _BLK, Hh), lambda i: (0, i, 0)),
                  pl.BlockSpec((H, H), lambda i: (0, 0)),
                  pl.BlockSpec((1, H), lambda i: (0, 0)),
                  pl.BlockSpec((H, H), lambda i: (0, 0)),
                  pl.BlockSpec((1, H), lambda i: (0, 0))],
        out_specs=[pl.BlockSpec((_BLK, H), lambda i: (i, 0)),
                   pl.BlockSpec((_BLK, H), lambda i: (i, 0))],
        out_shape=[jax.ShapeDtypeStruct((Nn, H), jnp.float32),
                   jax.ShapeDtypeStruct((Nn, H), jnp.float32)],
    )(h, gW, gb.reshape(1, -1), lW, lb.reshape(1, -1))


def _mix(p, gate, lin, gcW, gcb, gc3W=None):
    _, Nn, Hh = p.shape
    H = 2 * Hh

    def compute(p_ref, g_ref, l_ref, gcw, gcb_):
        agg = jnp.concatenate([p_ref[0], p_ref[1]], axis=1)
        conv = jnp.tanh(
            jnp.dot(agg, gcw[...], preferred_element_type=jnp.float32)
            + gcb_[...])
        return g_ref[...] * conv + (1.0 - g_ref[...]) * l_ref[...]

    in_specs = [
        pl.BlockSpec((2, _BLK, Hh), lambda i: (0, i, 0)),
        pl.BlockSpec((_BLK, H), lambda i: (i, 0)),
        pl.BlockSpec((_BLK, H), lambda i: (i, 0)),
        pl.BlockSpec((H, H), lambda i: (0, 0)),
        pl.BlockSpec((1, H), lambda i: (0, 0)),
    ]
    args = [p, gate, lin, gcW, gcb.reshape(1, -1)]

    if gc3W is None:
        def kern_base(p_ref, g_ref, l_ref, gcw, gcb_, o_ref):
            hn = compute(p_ref, g_ref, l_ref, gcw, gcb_)
            o_ref[0] = hn[:, :Hh]
            o_ref[1] = hn[:, Hh:]

        return pl.pallas_call(
            kern_base,
            grid=(Nn // _BLK,),
            in_specs=in_specs,
            out_specs=pl.BlockSpec((2, _BLK, Hh), lambda i: (0, i, 0)),
            out_shape=jax.ShapeDtypeStruct((2, Nn, Hh), jnp.float32),
        )(*args)

    Kc = gc3W.shape[1]

    def kern_fused(p_ref, g_ref, l_ref, gcw, gcb_, g3w, o_ref, s_ref):
        hn = compute(p_ref, g_ref, l_ref, gcw, gcb_)
        o_ref[0] = hn[:, :Hh]
        o_ref[1] = hn[:, Hh:]
        s_ref[...] = jnp.dot(hn, g3w[...], preferred_element_type=jnp.float32)

    return pl.pallas_call(
        kern_fused,
        grid=(Nn // _BLK,),
        in_specs=in_specs + [pl.BlockSpec((H, Kc), lambda i: (0, 0))],
        out_specs=[pl.BlockSpec((2, _BLK, Hh), lambda i: (0, i, 0)),
                   pl.BlockSpec((_BLK, Kc), lambda i: (i, 0))],
        out_shape=[jax.ShapeDtypeStruct((2, Nn, Hh), jnp.float32),
                   jax.ShapeDtypeStruct((Nn, Kc), jnp.float32)],
    )(*args, gc3W)


def _final_logsoftmax(p3, b3):
    _, Nn, Kc = p3.shape

    def kern(p_ref, b_ref, o_ref):
        v = p_ref[0] + p_ref[1] + b_ref[...]
        m = jnp.max(v, axis=1, keepdims=True)
        lse = jnp.log(jnp.sum(jnp.exp(v - m), axis=1, keepdims=True)) + m
        o_ref[...] = v - lse

    return pl.pallas_call(
        kern,
        grid=(Nn // _BLK,),
        in_specs=[pl.BlockSpec((NC, _BLK, Kc), lambda i: (0, i, 0)),
                  pl.BlockSpec((1, Kc), lambda i: (0, 0))],
        out_specs=pl.BlockSpec((_BLK, Kc), lambda i: (i, 0)),
        out_shape=jax.ShapeDtypeStruct((Nn, Kc), jnp.float32),
    )(p3, b3.reshape(1, -1))


def kernel(x, edge_index, W1, b1, hw1_gc_W, hw1_gc_b, hw1_gate_W, hw1_gate_b,
           hw1_lin_W, hw1_lin_b, hw2_gc_W, hw2_gc_b, hw2_gate_W, hw2_gate_b,
           hw2_lin_W, hw2_lin_b, gc3_W, gc3_b):
    E = edge_index.shape[1]
    src = edge_index[0].reshape(E // 128, 128)
    dst = edge_index[1].reshape(E // 128, 128)

    linear = _mm_act(x, W1, b1)

    p1 = _spmm(linear, src, dst, feat_split=True)
    gate1, lin1 = _gatelin(linear, hw1_gate_W, hw1_gate_b,
                           hw1_lin_W, hw1_lin_b)
    conv1 = _mix(p1, gate1, lin1, hw1_gc_W, hw1_gc_b)

    p2 = _spmm(conv1, src, dst, feat_split=True)
    gate2, lin2 = _gatelin(conv1, hw2_gate_W, hw2_gate_b,
                           hw2_lin_W, hw2_lin_b)
    conv2, support3 = _mix(p2, gate2, lin2, hw2_gc_W, hw2_gc_b, gc3W=gc3_W)

    p3 = _spmm(support3, src, dst, feat_split=False)
    return _final_logsoftmax(p3, gc3_b)


# speedup vs baseline: 1.0430x; 1.0430x over previous
import functools

import jax
import jax.numpy as jnp
from jax import lax
from jax.experimental import pallas as pl
from jax.experimental.pallas import tpu as pltpu
from jax.experimental.pallas import tpu_sc as plsc

NC = 2
NS = 16


def _spmm(h, src2d, dst2d, feat_split):
    if feat_split:
        _, N, W = h.shape
    else:
        N, W = h.shape
    R = src2d.shape[0]
    CH = 128
    TPW = NS if feat_split else NC * NS
    RPTile = R // TPW
    XTRA = R - TPW * RPTile
    K = 6
    NJ = RPTile // K
    assert RPTile == NJ * K and NJ >= 2
    RPT = (N // NS) & ~7
    REM = N - NS * RPT
    nzc = RPT // CH
    zrem = RPT - nzc * CH

    mesh = plsc.VectorSubcoreMesh(core_axis_name="c", subcore_axis_name="s",
                                  num_cores=NC, num_subcores=NS)

    def body(h_hbm, s_hbm, d_hbm, z_hbm, out_hbm, acc, srcA, dstA,
             r0b, r1b, r2b, r3b, r4b, r5b, xsrc, xdst,
             g0, g1, g2, g3, g4, g5, s0, s1, s2, s3, s4, s5):
        c = lax.axis_index("c")
        s = lax.axis_index("s")
        rows = [r0b, r1b, r2b, r3b, r4b, r5b]
        gsem = [g0, g1, g2, g3, g4, g5]
        ssem = [s0, s1, s2, s3, s4, s5]
        table = h_hbm.at[c] if feat_split else h_hbm

        pltpu.sync_copy(z_hbm, rows[0])
        r0 = s * RPT
        zcopies = []
        for k in range(nzc):
            zcopies.append(pltpu.async_copy(
                rows[0], acc.at[pl.ds(r0 + k * CH, CH)], gsem[1]))
        if zrem:
            zcopies.append(pltpu.async_copy(
                rows[0].at[pl.ds(0, zrem)],
                acc.at[pl.ds(r0 + nzc * CH, zrem)], gsem[1]))

        rowbase = (s if feat_split else c * NS + s) * RPTile
        pltpu.async_copy(s_hbm.at[pl.ds(rowbase, RPTile)], srcA, gsem[2])
        pltpu.async_copy(d_hbm.at[pl.ds(rowbase, RPTile)], dstA, gsem[3])
        for cp in zcopies:
            cp.wait()
        if REM:
            @pl.when(s == 0)
            def _():
                pltpu.sync_copy(rows[0].at[pl.ds(0, REM)],
                                acc.at[pl.ds(NS * RPT, REM)])
        pltpu.make_async_copy(s_hbm.at[pl.ds(rowbase, RPTile)], srcA,
                              gsem[2]).wait()
        pltpu.make_async_copy(d_hbm.at[pl.ds(rowbase, RPTile)], dstA,
                              gsem[3]).wait()
        plsc.subcore_barrier()

        def gstart(t, v):
            pltpu.async_copy(table.at[srcA.at[t]], rows[v], gsem[v])

        def gwait(t, v):
            pltpu.make_async_copy(table.at[srcA.at[t]], rows[v],
                                  gsem[v]).wait()

        def sstart(t, v):
            pltpu.async_copy(rows[v], acc.at[dstA.at[t]], ssem[v], add=True)

        def swait(t, v):
            pltpu.make_async_copy(rows[v], acc.at[dstA.at[t]],
                                  ssem[v]).wait()

        for v in range(K):
            gstart(v, v)
            if v >= 1:
                gwait(v - 1, v - 1)
                sstart(v - 1, v - 1)

        def rev(j, _):
            for v in range(K):
                t = K * j + v
                pv = (v - 1) % K
                swait(t - K, v)
                gstart(t, v)
                gwait(t - 1, pv)
                sstart(t - 1, pv)
            return 0
        lax.fori_loop(1, NJ, rev, 0)

        last = RPTile - 1
        gwait(last, K - 1)
        sstart(last, K - 1)
        for v in range(K):
            swait(RPTile - K + v, v)

        if XTRA:
            u = s if feat_split else c * NS + s

            @pl.when(u < XTRA)
            def _():
                xr = TPW * RPTile + u
                pltpu.sync_copy(s_hbm.at[xr], xsrc)
                pltpu.sync_copy(d_hbm.at[xr], xdst)
                pltpu.async_copy(table.at[xsrc], rows[0], gsem[0]).wait()
                pltpu.sync_copy(rows[0], acc.at[xdst], add=True)
        plsc.subcore_barrier()

        pltpu.sync_copy(acc.at[pl.ds(r0, RPT)],
                        out_hbm.at[c, pl.ds(r0, RPT)])
        if REM:
            @pl.when(s == 0)
            def _():
                pltpu.sync_copy(acc.at[pl.ds(NS * RPT, REM)],
                                out_hbm.at[c, pl.ds(NS * RPT, REM)])

    f = pl.kernel(
        body,
        out_type=jax.ShapeDtypeStruct((NC, N, W), jnp.float32),
        mesh=mesh,
        scratch_types=(
            [pltpu.VMEM_SHARED((N, W), jnp.float32),
             pltpu.VMEM((RPTile, CH), jnp.int32),
             pltpu.VMEM((RPTile, CH), jnp.int32)]
            + [pltpu.VMEM((CH, W), jnp.float32) for _ in range(K)]
            + [pltpu.VMEM((CH,), jnp.int32),
               pltpu.VMEM((CH,), jnp.int32)]
            + [pltpu.SemaphoreType.DMA for _ in range(2 * K)]
        ),
        compiler_params=pltpu.CompilerParams(use_tc_tiling_on_sc=False),
    )
    return f(h, src2d, dst2d, jnp.zeros((CH, W), jnp.float32))


_BLK = 2000


def _mm_act(x, w, b):
    Nn, F = x.shape
    H = w.shape[1]
    Hh = H // 2

    def kern(x_ref, w_ref, b_ref, o_ref):
        v = jnp.tanh(
            jnp.dot(x_ref[...], w_ref[...],
                    preferred_element_type=jnp.float32) + b_ref[...])
        o_ref[0] = v[:, :Hh]
        o_ref[1] = v[:, Hh:]

    return pl.pallas_call(
        kern,
        grid=(Nn // _BLK,),
        in_specs=[pl.BlockSpec((_BLK, F), lambda i: (i, 0)),
                  pl.BlockSpec((F, H), lambda i: (0, 0)),
                  pl.BlockSpec((1, H), lambda i: (0, 0))],
        out_specs=pl.BlockSpec((2, _BLK, Hh), lambda i: (0, i, 0)),
        out_shape=jax.ShapeDtypeStruct((2, Nn, Hh), jnp.float32),
    )(x, w, b.reshape(1, -1))


def _highway(h, p, gcW, gcb, gW, gb, lW, lb, gc3W=None):
    _, Nn, Hh = h.shape
    H = 2 * Hh

    def compute(h_ref, p_ref, gcw, gcb_, gw, gb_, lw, lb_):
        hh = jnp.concatenate([h_ref[0], h_ref[1]], axis=1)
        agg = jnp.concatenate([p_ref[0], p_ref[1]], axis=1)
        conv = jnp.tanh(
            jnp.dot(agg, gcw[...], preferred_element_type=jnp.float32)
            + gcb_[...])
        gate = jax.nn.sigmoid(
            jnp.dot(hh, gw[...], preferred_element_type=jnp.float32)
            + gb_[...])
        linv = (jnp.dot(hh, lw[...],
                        preferred_element_type=jnp.float32) + lb_[...])
        return gate * conv + (1.0 - gate) * linv

    in_specs = [
        pl.BlockSpec((2, _BLK, Hh), lambda i: (0, i, 0)),
        pl.BlockSpec((2, _BLK, Hh), lambda i: (0, i, 0)),
        pl.BlockSpec((H, H), lambda i: (0, 0)),
        pl.BlockSpec((1, H), lambda i: (0, 0)),
        pl.BlockSpec((H, H), lambda i: (0, 0)),
        pl.BlockSpec((1, H), lambda i: (0, 0)),
        pl.BlockSpec((H, H), lambda i: (0, 0)),
        pl.BlockSpec((1, H), lambda i: (0, 0)),
    ]
    args = [h, p, gcW, gcb.reshape(1, -1), gW, gb.reshape(1, -1),
            lW, lb.reshape(1, -1)]

    if gc3W is None:
        def kern_base(h_ref, p_ref, gcw, gcb_, gw, gb_, lw, lb_, o_ref):
            hn = compute(h_ref, p_ref, gcw, gcb_, gw, gb_, lw, lb_)
            o_ref[0] = hn[:, :Hh]
            o_ref[1] = hn[:, Hh:]

        return pl.pallas_call(
            kern_base,
            grid=(Nn // _BLK,),
            in_specs=in_specs,
            out_specs=pl.BlockSpec((2, _BLK, Hh), lambda i: (0, i, 0)),
            out_shape=jax.ShapeDtypeStruct((2, Nn, Hh), jnp.float32),
        )(*args)

    Kc = gc3W.shape[1]

    def kern_fused(h_ref, p_ref, gcw, gcb_, gw, gb_, lw, lb_, g3w,
                   o_ref, s_ref):
        hn = compute(h_ref, p_ref, gcw, gcb_, gw, gb_, lw, lb_)
        o_ref[0] = hn[:, :Hh]
        o_ref[1] = hn[:, Hh:]
        s_ref[...] = jnp.dot(hn, g3w[...], preferred_element_type=jnp.float32)

    return pl.pallas_call(
        kern_fused,
        grid=(Nn // _BLK,),
        in_specs=in_specs + [pl.BlockSpec((H, Kc), lambda i: (0, 0))],
        out_specs=[pl.BlockSpec((2, _BLK, Hh), lambda i: (0, i, 0)),
                   pl.BlockSpec((_BLK, Kc), lambda i: (i, 0))],
        out_shape=[jax.ShapeDtypeStruct((2, Nn, Hh), jnp.float32),
                   jax.ShapeDtypeStruct((Nn, Kc), jnp.float32)],
    )(*args, gc3W)


def _final_logsoftmax(p3, b3):
    _, Nn, Kc = p3.shape

    def kern(p_ref, b_ref, o_ref):
        v = p_ref[0] + p_ref[1] + b_ref[...]
        m = jnp.max(v, axis=1, keepdims=True)
        lse = jnp.log(jnp.sum(jnp.exp(v - m), axis=1, keepdims=True)) + m
        o_ref[...] = v - lse

    return pl.pallas_call(
        kern,
        grid=(Nn // _BLK,),
        in_specs=[pl.BlockSpec((NC, _BLK, Kc), lambda i: (0, i, 0)),
                  pl.BlockSpec((1, Kc), lambda i: (0, 0))],
        out_specs=pl.BlockSpec((_BLK, Kc), lambda i: (i, 0)),
        out_shape=jax.ShapeDtypeStruct((Nn, Kc), jnp.float32),
    )(p3, b3.reshape(1, -1))


def kernel(x, edge_index, W1, b1, hw1_gc_W, hw1_gc_b, hw1_gate_W, hw1_gate_b,
           hw1_lin_W, hw1_lin_b, hw2_gc_W, hw2_gc_b, hw2_gate_W, hw2_gate_b,
           hw2_lin_W, hw2_lin_b, gc3_W, gc3_b):
    E = edge_index.shape[1]
    src = edge_index[0].reshape(E // 128, 128)
    dst = edge_index[1].reshape(E // 128, 128)

    linear = _mm_act(x, W1, b1)

    p1 = _spmm(linear, src, dst, feat_split=True)
    conv1 = _highway(linear, p1, hw1_gc_W, hw1_gc_b, hw1_gate_W, hw1_gate_b,
                     hw1_lin_W, hw1_lin_b)

    p2 = _spmm(conv1, src, dst, feat_split=True)
    conv2, support3 = _highway(conv1, p2, hw2_gc_W, hw2_gc_b, hw2_gate_W,
                               hw2_gate_b, hw2_lin_W, hw2_lin_b, gc3W=gc3_W)

    p3 = _spmm(support3, src, dst, feat_split=False)
    return _final_logsoftmax(p3, gc3_b)


# speedup vs baseline: 1.2350x; 1.1842x over previous
import functools

import jax
import jax.numpy as jnp
from jax import lax
from jax.experimental import pallas as pl
from jax.experimental.pallas import tpu as pltpu
from jax.experimental.pallas import tpu_sc as plsc

NC = 2
NS = 16


def _spmm(h, src2d, dst2d, feat_split):
    if feat_split:
        _, N, W = h.shape
    else:
        N, W = h.shape
    R = src2d.shape[0]
    CH = 128
    TPW = NS if feat_split else NC * NS
    RPTile = R // TPW
    XTRA = R - TPW * RPTile
    K = 6
    NJ = RPTile // K
    assert RPTile == NJ * K and NJ >= 2
    RPT = (N // NS) & ~7
    REM = N - NS * RPT
    nzc = RPT // CH
    zrem = RPT - nzc * CH

    mesh = plsc.VectorSubcoreMesh(core_axis_name="c", subcore_axis_name="s",
                                  num_cores=NC, num_subcores=NS)

    def body(h_hbm, s_hbm, d_hbm, z_hbm, out_hbm, acc, srcA, dstA,
             r0b, r1b, r2b, r3b, r4b, r5b, xsrc, xdst,
             g0, g1, g2, g3, g4, g5, s0, s1, s2, s3, s4, s5):
        c = lax.axis_index("c")
        s = lax.axis_index("s")
        rows = [r0b, r1b, r2b, r3b, r4b, r5b]
        gsem = [g0, g1, g2, g3, g4, g5]
        ssem = [s0, s1, s2, s3, s4, s5]
        table = h_hbm.at[c] if feat_split else h_hbm

        pltpu.sync_copy(z_hbm, rows[0])
        r0 = s * RPT
        zcopies = []
        for k in range(nzc):
            zcopies.append(pltpu.async_copy(
                rows[0], acc.at[pl.ds(r0 + k * CH, CH)], gsem[1]))
        if zrem:
            zcopies.append(pltpu.async_copy(
                rows[0].at[pl.ds(0, zrem)],
                acc.at[pl.ds(r0 + nzc * CH, zrem)], gsem[1]))

        rowbase = (s if feat_split else c * NS + s) * RPTile
        pltpu.async_copy(s_hbm.at[pl.ds(rowbase, RPTile)], srcA, gsem[2])
        pltpu.async_copy(d_hbm.at[pl.ds(rowbase, RPTile)], dstA, gsem[3])
        for cp in zcopies:
            cp.wait()
        if REM:
            @pl.when(s == 0)
            def _():
                pltpu.sync_copy(rows[0].at[pl.ds(0, REM)],
                                acc.at[pl.ds(NS * RPT, REM)])
        pltpu.make_async_copy(s_hbm.at[pl.ds(rowbase, RPTile)], srcA,
                              gsem[2]).wait()
        pltpu.make_async_copy(d_hbm.at[pl.ds(rowbase, RPTile)], dstA,
                              gsem[3]).wait()
        plsc.subcore_barrier()

        def gstart(t, v):
            pltpu.async_copy(table.at[srcA.at[t]], rows[v], gsem[v])

        def gwait(t, v):
            pltpu.make_async_copy(table.at[srcA.at[t]], rows[v],
                                  gsem[v]).wait()

        def sstart(t, v):
            pltpu.async_copy(rows[v], acc.at[dstA.at[t]], ssem[v], add=True)

        def swait(t, v):
            pltpu.make_async_copy(rows[v], acc.at[dstA.at[t]],
                                  ssem[v]).wait()

        D = 3
        for v in range(K):
            gstart(v, v)
            if v >= D:
                gwait(v - D, v - D)
                sstart(v - D, v - D)

        def rev(j, _):
            for v in range(K):
                t = K * j + v
                pv = (v - D) % K
                swait(t - K, v)
                gstart(t, v)
                gwait(t - D, pv)
                sstart(t - D, pv)
            return 0
        lax.fori_loop(1, NJ, rev, 0)

        for d in range(D, 0, -1):
            t = RPTile - d
            vv = (K - d) % K
            gwait(t, vv)
            sstart(t, vv)
        for v in range(K):
            swait(RPTile - K + v, v)

        if XTRA:
            u = s if feat_split else c * NS + s

            @pl.when(u < XTRA)
            def _():
                xr = TPW * RPTile + u
                pltpu.sync_copy(s_hbm.at[xr], xsrc)
                pltpu.sync_copy(d_hbm.at[xr], xdst)
                pltpu.async_copy(table.at[xsrc], rows[0], gsem[0]).wait()
                pltpu.sync_copy(rows[0], acc.at[xdst], add=True)
        plsc.subcore_barrier()

        pltpu.sync_copy(acc.at[pl.ds(r0, RPT)],
                        out_hbm.at[c, pl.ds(r0, RPT)])
        if REM:
            @pl.when(s == 0)
            def _():
                pltpu.sync_copy(acc.at[pl.ds(NS * RPT, REM)],
                                out_hbm.at[c, pl.ds(NS * RPT, REM)])

    f = pl.kernel(
        body,
        out_type=jax.ShapeDtypeStruct((NC, N, W), jnp.float32),
        mesh=mesh,
        scratch_types=(
            [pltpu.VMEM_SHARED((N, W), jnp.float32),
             pltpu.VMEM((RPTile, CH), jnp.int32),
             pltpu.VMEM((RPTile, CH), jnp.int32)]
            + [pltpu.VMEM((CH, W), jnp.float32) for _ in range(K)]
            + [pltpu.VMEM((CH,), jnp.int32),
               pltpu.VMEM((CH,), jnp.int32)]
            + [pltpu.SemaphoreType.DMA for _ in range(2 * K)]
        ),
        compiler_params=pltpu.CompilerParams(use_tc_tiling_on_sc=False),
    )
    return f(h, src2d, dst2d, jnp.zeros((CH, W), jnp.float32))


_BLK = 2000


def _mm_act(x, w, b):
    Nn, F = x.shape
    H = w.shape[1]
    Hh = H // 2

    def kern(x_ref, w_ref, b_ref, o_ref):
        v = jnp.tanh(
            jnp.dot(x_ref[...], w_ref[...],
                    preferred_element_type=jnp.float32) + b_ref[...])
        o_ref[0] = v[:, :Hh]
        o_ref[1] = v[:, Hh:]

    return pl.pallas_call(
        kern,
        grid=(Nn // _BLK,),
        in_specs=[pl.BlockSpec((_BLK, F), lambda i: (i, 0)),
                  pl.BlockSpec((F, H), lambda i: (0, 0)),
                  pl.BlockSpec((1, H), lambda i: (0, 0))],
        out_specs=pl.BlockSpec((2, _BLK, Hh), lambda i: (0, i, 0)),
        out_shape=jax.ShapeDtypeStruct((2, Nn, Hh), jnp.float32),
    )(x, w, b.reshape(1, -1))


def _highway(h, p, gcW, gcb, gW, gb, lW, lb, gc3W=None):
    _, Nn, Hh = h.shape
    H = 2 * Hh

    def compute(h_ref, p_ref, gcw, gcb_, gw, gb_, lw, lb_):
        hh = jnp.concatenate([h_ref[0], h_ref[1]], axis=1)
        agg = jnp.concatenate([p_ref[0], p_ref[1]], axis=1)
        conv = jnp.tanh(
            jnp.dot(agg, gcw[...], preferred_element_type=jnp.float32)
            + gcb_[...])
        gate = jax.nn.sigmoid(
            jnp.dot(hh, gw[...], preferred_element_type=jnp.float32)
            + gb_[...])
        linv = (jnp.dot(hh, lw[...],
                        preferred_element_type=jnp.float32) + lb_[...])
        return gate * conv + (1.0 - gate) * linv

    in_specs = [
        pl.BlockSpec((2, _BLK, Hh), lambda i: (0, i, 0)),
        pl.BlockSpec((2, _BLK, Hh), lambda i: (0, i, 0)),
        pl.BlockSpec((H, H), lambda i: (0, 0)),
        pl.BlockSpec((1, H), lambda i: (0, 0)),
        pl.BlockSpec((H, H), lambda i: (0, 0)),
        pl.BlockSpec((1, H), lambda i: (0, 0)),
        pl.BlockSpec((H, H), lambda i: (0, 0)),
        pl.BlockSpec((1, H), lambda i: (0, 0)),
    ]
    args = [h, p, gcW, gcb.reshape(1, -1), gW, gb.reshape(1, -1),
            lW, lb.reshape(1, -1)]

    if gc3W is None:
        def kern_base(h_ref, p_ref, gcw, gcb_, gw, gb_, lw, lb_, o_ref):
            hn = compute(h_ref, p_ref, gcw, gcb_, gw, gb_, lw, lb_)
            o_ref[0] = hn[:, :Hh]
            o_ref[1] = hn[:, Hh:]

        return pl.pallas_call(
            kern_base,
            grid=(Nn // _BLK,),
            in_specs=in_specs,
            out_specs=pl.BlockSpec((2, _BLK, Hh), lambda i: (0, i, 0)),
            out_shape=jax.ShapeDtypeStruct((2, Nn, Hh), jnp.float32),
        )(*args)

    Kc = gc3W.shape[1]

    def kern_fused(h_ref, p_ref, gcw, gcb_, gw, gb_, lw, lb_, g3w,
                   o_ref, s_ref):
        hn = compute(h_ref, p_ref, gcw, gcb_, gw, gb_, lw, lb_)
        o_ref[0] = hn[:, :Hh]
        o_ref[1] = hn[:, Hh:]
        s_ref[...] = jnp.dot(hn, g3w[...], preferred_element_type=jnp.float32)

    return pl.pallas_call(
        kern_fused,
        grid=(Nn // _BLK,),
        in_specs=in_specs + [pl.BlockSpec((H, Kc), lambda i: (0, 0))],
        out_specs=[pl.BlockSpec((2, _BLK, Hh), lambda i: (0, i, 0)),
                   pl.BlockSpec((_BLK, Kc), lambda i: (i, 0))],
        out_shape=[jax.ShapeDtypeStruct((2, Nn, Hh), jnp.float32),
                   jax.ShapeDtypeStruct((Nn, Kc), jnp.float32)],
    )(*args, gc3W)


def _final_logsoftmax(p3, b3):
    _, Nn, Kc = p3.shape

    def kern(p_ref, b_ref, o_ref):
        v = p_ref[0] + p_ref[1] + b_ref[...]
        m = jnp.max(v, axis=1, keepdims=True)
        lse = jnp.log(jnp.sum(jnp.exp(v - m), axis=1, keepdims=True)) + m
        o_ref[...] = v - lse

    return pl.pallas_call(
        kern,
        grid=(Nn // _BLK,),
        in_specs=[pl.BlockSpec((NC, _BLK, Kc), lambda i: (0, i, 0)),
                  pl.BlockSpec((1, Kc), lambda i: (0, 0))],
        out_specs=pl.BlockSpec((_BLK, Kc), lambda i: (i, 0)),
        out_shape=jax.ShapeDtypeStruct((Nn, Kc), jnp.float32),
    )(p3, b3.reshape(1, -1))


def kernel(x, edge_index, W1, b1, hw1_gc_W, hw1_gc_b, hw1_gate_W, hw1_gate_b,
           hw1_lin_W, hw1_lin_b, hw2_gc_W, hw2_gc_b, hw2_gate_W, hw2_gate_b,
           hw2_lin_W, hw2_lin_b, gc3_W, gc3_b):
    E = edge_index.shape[1]
    src = edge_index[0].reshape(E // 128, 128)
    dst = edge_index[1].reshape(E // 128, 128)

    linear = _mm_act(x, W1, b1)

    p1 = _spmm(linear, src, dst, feat_split=True)
    conv1 = _highway(linear, p1, hw1_gc_W, hw1_gc_b, hw1_gate_W, hw1_gate_b,
                     hw1_lin_W, hw1_lin_b)

    p2 = _spmm(conv1, src, dst, feat_split=True)
    conv2, support3 = _highway(conv1, p2, hw2_gc_W, hw2_gc_b, hw2_gate_W,
                               hw2_gate_b, hw2_lin_W, hw2_lin_b, gc3W=gc3_W)

    p3 = _spmm(support3, src, dst, feat_split=False)
    return _final_logsoftmax(p3, gc3_b)


# speedup vs baseline: 1.2466x; 1.0094x over previous
import functools

import jax
import jax.numpy as jnp
from jax import lax
from jax.experimental import pallas as pl
from jax.experimental.pallas import tpu as pltpu
from jax.experimental.pallas import tpu_sc as plsc

NC = 2
NS = 16


def _spmm(h, src2d, dst2d, feat_split):
    if feat_split:
        _, N, W = h.shape
    else:
        N, W = h.shape
    R = src2d.shape[0]
    CH = 128
    TPW = NS if feat_split else NC * NS
    RPTile = R // TPW
    XTRA = R - TPW * RPTile
    K = 6
    NJ = RPTile // K
    assert RPTile == NJ * K and NJ >= 2
    RPT = (N // NS) & ~7
    REM = N - NS * RPT
    nzc = RPT // CH
    zrem = RPT - nzc * CH

    mesh = plsc.VectorSubcoreMesh(core_axis_name="c", subcore_axis_name="s",
                                  num_cores=NC, num_subcores=NS)

    def body(h_hbm, s_hbm, d_hbm, z_hbm, out_hbm, acc, srcA, dstA,
             r0b, r1b, r2b, r3b, r4b, r5b, xsrc, xdst,
             g0, g1, g2, g3, g4, g5, s0, s1, s2, s3, s4, s5):
        c = lax.axis_index("c")
        s = lax.axis_index("s")
        rows = [r0b, r1b, r2b, r3b, r4b, r5b]
        gsem = [g0, g1, g2, g3, g4, g5]
        ssem = [s0, s1, s2, s3, s4, s5]
        table = h_hbm.at[c] if feat_split else h_hbm

        pltpu.sync_copy(z_hbm, rows[0])
        r0 = s * RPT
        zcopies = []
        for k in range(nzc):
            zcopies.append(pltpu.async_copy(
                rows[0], acc.at[pl.ds(r0 + k * CH, CH)], gsem[1]))
        if zrem:
            zcopies.append(pltpu.async_copy(
                rows[0].at[pl.ds(0, zrem)],
                acc.at[pl.ds(r0 + nzc * CH, zrem)], gsem[1]))

        rowbase = (s if feat_split else c * NS + s) * RPTile
        pltpu.async_copy(s_hbm.at[pl.ds(rowbase, RPTile)], srcA, gsem[2])
        pltpu.async_copy(d_hbm.at[pl.ds(rowbase, RPTile)], dstA, gsem[3])
        for cp in zcopies:
            cp.wait()
        if REM:
            @pl.when(s == 0)
            def _():
                pltpu.sync_copy(rows[0].at[pl.ds(0, REM)],
                                acc.at[pl.ds(NS * RPT, REM)])
        pltpu.make_async_copy(s_hbm.at[pl.ds(rowbase, RPTile)], srcA,
                              gsem[2]).wait()
        pltpu.make_async_copy(d_hbm.at[pl.ds(rowbase, RPTile)], dstA,
                              gsem[3]).wait()
        plsc.subcore_barrier()

        def gstart(t, v):
            pltpu.async_copy(table.at[srcA.at[t]], rows[v], gsem[v])

        def gwait(t, v):
            pltpu.make_async_copy(table.at[srcA.at[t]], rows[v],
                                  gsem[v]).wait()

        def sstart(t, v):
            pltpu.async_copy(rows[v], acc.at[dstA.at[t]], ssem[v], add=True)

        def swait(t, v):
            pltpu.make_async_copy(rows[v], acc.at[dstA.at[t]],
                                  ssem[v]).wait()

        D = 5
        for v in range(K):
            gstart(v, v)
            if v >= D:
                gwait(v - D, v - D)
                sstart(v - D, v - D)

        def rev(j, _):
            for v in range(K):
                t = K * j + v
                pv = (v - D) % K
                swait(t - K, v)
                gstart(t, v)
                gwait(t - D, pv)
                sstart(t - D, pv)
            return 0
        lax.fori_loop(1, NJ, rev, 0)

        for d in range(D, 0, -1):
            t = RPTile - d
            vv = (K - d) % K
            gwait(t, vv)
            sstart(t, vv)
        for v in range(K):
            swait(RPTile - K + v, v)

        if XTRA:
            u = s if feat_split else c * NS + s

            @pl.when(u < XTRA)
            def _():
                xr = TPW * RPTile + u
                pltpu.sync_copy(s_hbm.at[xr], xsrc)
                pltpu.sync_copy(d_hbm.at[xr], xdst)
                pltpu.async_copy(table.at[xsrc], rows[0], gsem[0]).wait()
                pltpu.sync_copy(rows[0], acc.at[xdst], add=True)
        plsc.subcore_barrier()

        pltpu.sync_copy(acc.at[pl.ds(r0, RPT)],
                        out_hbm.at[c, pl.ds(r0, RPT)])
        if REM:
            @pl.when(s == 0)
            def _():
                pltpu.sync_copy(acc.at[pl.ds(NS * RPT, REM)],
                                out_hbm.at[c, pl.ds(NS * RPT, REM)])

    f = pl.kernel(
        body,
        out_type=jax.ShapeDtypeStruct((NC, N, W), jnp.float32),
        mesh=mesh,
        scratch_types=(
            [pltpu.VMEM_SHARED((N, W), jnp.float32),
             pltpu.VMEM((RPTile, CH), jnp.int32),
             pltpu.VMEM((RPTile, CH), jnp.int32)]
            + [pltpu.VMEM((CH, W), jnp.float32) for _ in range(K)]
            + [pltpu.VMEM((CH,), jnp.int32),
               pltpu.VMEM((CH,), jnp.int32)]
            + [pltpu.SemaphoreType.DMA for _ in range(2 * K)]
        ),
        compiler_params=pltpu.CompilerParams(use_tc_tiling_on_sc=False),
    )
    return f(h, src2d, dst2d, jnp.zeros((CH, W), jnp.float32))


_BLK = 2000


def _mm_act(x, w, b):
    Nn, F = x.shape
    H = w.shape[1]
    Hh = H // 2

    def kern(x_ref, w_ref, b_ref, o_ref):
        v = jnp.tanh(
            jnp.dot(x_ref[...], w_ref[...],
                    preferred_element_type=jnp.float32) + b_ref[...])
        o_ref[0] = v[:, :Hh]
        o_ref[1] = v[:, Hh:]

    return pl.pallas_call(
        kern,
        grid=(Nn // _BLK,),
        in_specs=[pl.BlockSpec((_BLK, F), lambda i: (i, 0)),
                  pl.BlockSpec((F, H), lambda i: (0, 0)),
                  pl.BlockSpec((1, H), lambda i: (0, 0))],
        out_specs=pl.BlockSpec((2, _BLK, Hh), lambda i: (0, i, 0)),
        out_shape=jax.ShapeDtypeStruct((2, Nn, Hh), jnp.float32),
    )(x, w, b.reshape(1, -1))


def _highway(h, p, gcW, gcb, gW, gb, lW, lb, gc3W=None):
    _, Nn, Hh = h.shape
    H = 2 * Hh

    def compute(h_ref, p_ref, gcw, gcb_, gw, gb_, lw, lb_):
        hh = jnp.concatenate([h_ref[0], h_ref[1]], axis=1)
        agg = jnp.concatenate([p_ref[0], p_ref[1]], axis=1)
        conv = jnp.tanh(
            jnp.dot(agg, gcw[...], preferred_element_type=jnp.float32)
            + gcb_[...])
        gate = jax.nn.sigmoid(
            jnp.dot(hh, gw[...], preferred_element_type=jnp.float32)
            + gb_[...])
        linv = (jnp.dot(hh, lw[...],
                        preferred_element_type=jnp.float32) + lb_[...])
        return gate * conv + (1.0 - gate) * linv

    in_specs = [
        pl.BlockSpec((2, _BLK, Hh), lambda i: (0, i, 0)),
        pl.BlockSpec((2, _BLK, Hh), lambda i: (0, i, 0)),
        pl.BlockSpec((H, H), lambda i: (0, 0)),
        pl.BlockSpec((1, H), lambda i: (0, 0)),
        pl.BlockSpec((H, H), lambda i: (0, 0)),
        pl.BlockSpec((1, H), lambda i: (0, 0)),
        pl.BlockSpec((H, H), lambda i: (0, 0)),
        pl.BlockSpec((1, H), lambda i: (0, 0)),
    ]
    args = [h, p, gcW, gcb.reshape(1, -1), gW, gb.reshape(1, -1),
            lW, lb.reshape(1, -1)]

    if gc3W is None:
        def kern_base(h_ref, p_ref, gcw, gcb_, gw, gb_, lw, lb_, o_ref):
            hn = compute(h_ref, p_ref, gcw, gcb_, gw, gb_, lw, lb_)
            o_ref[0] = hn[:, :Hh]
            o_ref[1] = hn[:, Hh:]

        return pl.pallas_call(
            kern_base,
            grid=(Nn // _BLK,),
            in_specs=in_specs,
            out_specs=pl.BlockSpec((2, _BLK, Hh), lambda i: (0, i, 0)),
            out_shape=jax.ShapeDtypeStruct((2, Nn, Hh), jnp.float32),
        )(*args)

    Kc = gc3W.shape[1]

    def kern_fused(h_ref, p_ref, gcw, gcb_, gw, gb_, lw, lb_, g3w,
                   o_ref, s_ref):
        hn = compute(h_ref, p_ref, gcw, gcb_, gw, gb_, lw, lb_)
        o_ref[0] = hn[:, :Hh]
        o_ref[1] = hn[:, Hh:]
        s_ref[...] = jnp.dot(hn, g3w[...], preferred_element_type=jnp.float32)

    return pl.pallas_call(
        kern_fused,
        grid=(Nn // _BLK,),
        in_specs=in_specs + [pl.BlockSpec((H, Kc), lambda i: (0, 0))],
        out_specs=[pl.BlockSpec((2, _BLK, Hh), lambda i: (0, i, 0)),
                   pl.BlockSpec((_BLK, Kc), lambda i: (i, 0))],
        out_shape=[jax.ShapeDtypeStruct((2, Nn, Hh), jnp.float32),
                   jax.ShapeDtypeStruct((Nn, Kc), jnp.float32)],
    )(*args, gc3W)


def _final_logsoftmax(p3, b3):
    _, Nn, Kc = p3.shape

    def kern(p_ref, b_ref, o_ref):
        v = p_ref[0] + p_ref[1] + b_ref[...]
        m = jnp.max(v, axis=1, keepdims=True)
        lse = jnp.log(jnp.sum(jnp.exp(v - m), axis=1, keepdims=True)) + m
        o_ref[...] = v - lse

    return pl.pallas_call(
        kern,
        grid=(Nn // _BLK,),
        in_specs=[pl.BlockSpec((NC, _BLK, Kc), lambda i: (0, i, 0)),
                  pl.BlockSpec((1, Kc), lambda i: (0, 0))],
        out_specs=pl.BlockSpec((_BLK, Kc), lambda i: (i, 0)),
        out_shape=jax.ShapeDtypeStruct((Nn, Kc), jnp.float32),
    )(p3, b3.reshape(1, -1))


def kernel(x, edge_index, W1, b1, hw1_gc_W, hw1_gc_b, hw1_gate_W, hw1_gate_b,
           hw1_lin_W, hw1_lin_b, hw2_gc_W, hw2_gc_b, hw2_gate_W, hw2_gate_b,
           hw2_lin_W, hw2_lin_b, gc3_W, gc3_b):
    E = edge_index.shape[1]
    src = edge_index[0].reshape(E // 128, 128)
    dst = edge_index[1].reshape(E // 128, 128)

    linear = _mm_act(x, W1, b1)

    p1 = _spmm(linear, src, dst, feat_split=True)
    conv1 = _highway(linear, p1, hw1_gc_W, hw1_gc_b, hw1_gate_W, hw1_gate_b,
                     hw1_lin_W, hw1_lin_b)

    p2 = _spmm(conv1, src, dst, feat_split=True)
    conv2, support3 = _highway(conv1, p2, hw2_gc_W, hw2_gc_b, hw2_gate_W,
                               hw2_gate_b, hw2_lin_W, hw2_lin_b, gc3W=gc3_W)

    p3 = _spmm(support3, src, dst, feat_split=False)
    return _final_logsoftmax(p3, gc3_b)


# speedup vs baseline: 1.2467x; 1.0001x over previous
import functools

import jax
import jax.numpy as jnp
from jax import lax
from jax.experimental import pallas as pl
from jax.experimental.pallas import tpu as pltpu
from jax.experimental.pallas import tpu_sc as plsc

NC = 2
NS = 16


def _spmm(h, src2d, dst2d, feat_split):
    if feat_split:
        _, N, W = h.shape
    else:
        N, W = h.shape
    R = src2d.shape[0]
    CH = 128
    TPW = NS if feat_split else NC * NS
    RPTile = R // TPW
    XTRA = R - TPW * RPTile
    K = 6
    NJ = RPTile // K
    assert RPTile == NJ * K and NJ >= 2
    RPT = (N // NS) & ~7
    REM = N - NS * RPT
    nzc = RPT // CH
    zrem = RPT - nzc * CH

    mesh = plsc.VectorSubcoreMesh(core_axis_name="c", subcore_axis_name="s",
                                  num_cores=NC, num_subcores=NS)

    def body(h_hbm, s_hbm, d_hbm, z_hbm, out_hbm, acc, srcA, dstA,
             r0b, r1b, r2b, r3b, r4b, r5b, xsrc, xdst,
             g0, g1, g2, g3, g4, g5, s0, s1, s2, s3, s4, s5):
        c = lax.axis_index("c")
        s = lax.axis_index("s")
        rows = [r0b, r1b, r2b, r3b, r4b, r5b]
        gsem = [g0, g1, g2, g3, g4, g5]
        ssem = [s0, s1, s2, s3, s4, s5]
        table = h_hbm.at[c] if feat_split else h_hbm

        pltpu.sync_copy(z_hbm, rows[0])
        r0 = s * RPT
        zcopies = []
        for k in range(nzc):
            zcopies.append(pltpu.async_copy(
                rows[0], acc.at[pl.ds(r0 + k * CH, CH)], gsem[1]))
        if zrem:
            zcopies.append(pltpu.async_copy(
                rows[0].at[pl.ds(0, zrem)],
                acc.at[pl.ds(r0 + nzc * CH, zrem)], gsem[1]))

        rowbase = (s if feat_split else c * NS + s) * RPTile
        pltpu.async_copy(s_hbm.at[pl.ds(rowbase, RPTile)], srcA, gsem[2])
        pltpu.async_copy(d_hbm.at[pl.ds(rowbase, RPTile)], dstA, gsem[3])
        for cp in zcopies:
            cp.wait()
        if REM:
            @pl.when(s == 0)
            def _():
                pltpu.sync_copy(rows[0].at[pl.ds(0, REM)],
                                acc.at[pl.ds(NS * RPT, REM)])
        pltpu.make_async_copy(s_hbm.at[pl.ds(rowbase, RPTile)], srcA,
                              gsem[2]).wait()
        pltpu.make_async_copy(d_hbm.at[pl.ds(rowbase, RPTile)], dstA,
                              gsem[3]).wait()
        plsc.subcore_barrier()

        def gstart(t, v):
            pltpu.async_copy(table.at[srcA.at[t]], rows[v], gsem[v])

        def gwait(t, v):
            pltpu.make_async_copy(table.at[srcA.at[t]], rows[v],
                                  gsem[v]).wait()

        def sstart(t, v):
            pltpu.async_copy(rows[v], acc.at[dstA.at[t]], ssem[v], add=True)

        def swait(t, v):
            pltpu.make_async_copy(rows[v], acc.at[dstA.at[t]],
                                  ssem[v]).wait()

        D = 5
        for v in range(K):
            gstart(v, v)
            if v >= D:
                gwait(v - D, v - D)
                sstart(v - D, v - D)

        def rev(j, _):
            for v in range(K):
                t = K * j + v
                pv = (v - D) % K
                swait(t - K, v)
                gstart(t, v)
                gwait(t - D, pv)
                sstart(t - D, pv)
            return 0
        lax.fori_loop(1, NJ, rev, 0)

        for d in range(D, 0, -1):
            t = RPTile - d
            vv = (K - d) % K
            gwait(t, vv)
            sstart(t, vv)
        for v in range(K):
            swait(RPTile - K + v, v)

        if XTRA:
            u = s if feat_split else c * NS + s

            @pl.when(u < XTRA)
            def _():
                xr = TPW * RPTile + u
                pltpu.sync_copy(s_hbm.at[xr], xsrc)
                pltpu.sync_copy(d_hbm.at[xr], xdst)
                pltpu.async_copy(table.at[xsrc], rows[0], gsem[0]).wait()
                pltpu.sync_copy(rows[0], acc.at[xdst], add=True)
        plsc.subcore_barrier()

        pltpu.sync_copy(acc.at[pl.ds(r0, RPT)],
                        out_hbm.at[c, pl.ds(r0, RPT)])
        if REM:
            @pl.when(s == 0)
            def _():
                pltpu.sync_copy(acc.at[pl.ds(NS * RPT, REM)],
                                out_hbm.at[c, pl.ds(NS * RPT, REM)])

    f = pl.kernel(
        body,
        out_type=jax.ShapeDtypeStruct((NC, N, W), jnp.float32),
        mesh=mesh,
        scratch_types=(
            [pltpu.VMEM_SHARED((N, W), jnp.float32),
             pltpu.VMEM((RPTile, CH), jnp.int32),
             pltpu.VMEM((RPTile, CH), jnp.int32)]
            + [pltpu.VMEM((CH, W), jnp.float32) for _ in range(K)]
            + [pltpu.VMEM((CH,), jnp.int32),
               pltpu.VMEM((CH,), jnp.int32)]
            + [pltpu.SemaphoreType.DMA for _ in range(2 * K)]
        ),
        compiler_params=pltpu.CompilerParams(use_tc_tiling_on_sc=False,
                                             skip_device_barrier=True),
    )
    return f(h, src2d, dst2d, jnp.zeros((CH, W), jnp.float32))


_BLK = 2000


def _mm_act(x, w, b):
    Nn, F = x.shape
    H = w.shape[1]
    Hh = H // 2

    def kern(x_ref, w_ref, b_ref, o_ref):
        v = jnp.tanh(
            jnp.dot(x_ref[...], w_ref[...],
                    preferred_element_type=jnp.float32) + b_ref[...])
        o_ref[0] = v[:, :Hh]
        o_ref[1] = v[:, Hh:]

    return pl.pallas_call(
        kern,
        grid=(Nn // _BLK,),
        in_specs=[pl.BlockSpec((_BLK, F), lambda i: (i, 0)),
                  pl.BlockSpec((F, H), lambda i: (0, 0)),
                  pl.BlockSpec((1, H), lambda i: (0, 0))],
        out_specs=pl.BlockSpec((2, _BLK, Hh), lambda i: (0, i, 0)),
        out_shape=jax.ShapeDtypeStruct((2, Nn, Hh), jnp.float32),
    )(x, w, b.reshape(1, -1))


def _highway(h, p, gcW, gcb, gW, gb, lW, lb, gc3W=None):
    _, Nn, Hh = h.shape
    H = 2 * Hh

    def compute(h_ref, p_ref, gcw, gcb_, gw, gb_, lw, lb_):
        hh = jnp.concatenate([h_ref[0], h_ref[1]], axis=1)
        agg = jnp.concatenate([p_ref[0], p_ref[1]], axis=1)
        conv = jnp.tanh(
            jnp.dot(agg, gcw[...], preferred_element_type=jnp.float32)
            + gcb_[...])
        gate = jax.nn.sigmoid(
            jnp.dot(hh, gw[...], preferred_element_type=jnp.float32)
            + gb_[...])
        linv = (jnp.dot(hh, lw[...],
                        preferred_element_type=jnp.float32) + lb_[...])
        return gate * conv + (1.0 - gate) * linv

    in_specs = [
        pl.BlockSpec((2, _BLK, Hh), lambda i: (0, i, 0)),
        pl.BlockSpec((2, _BLK, Hh), lambda i: (0, i, 0)),
        pl.BlockSpec((H, H), lambda i: (0, 0)),
        pl.BlockSpec((1, H), lambda i: (0, 0)),
        pl.BlockSpec((H, H), lambda i: (0, 0)),
        pl.BlockSpec((1, H), lambda i: (0, 0)),
        pl.BlockSpec((H, H), lambda i: (0, 0)),
        pl.BlockSpec((1, H), lambda i: (0, 0)),
    ]
    args = [h, p, gcW, gcb.reshape(1, -1), gW, gb.reshape(1, -1),
            lW, lb.reshape(1, -1)]

    if gc3W is None:
        def kern_base(h_ref, p_ref, gcw, gcb_, gw, gb_, lw, lb_, o_ref):
            hn = compute(h_ref, p_ref, gcw, gcb_, gw, gb_, lw, lb_)
            o_ref[0] = hn[:, :Hh]
            o_ref[1] = hn[:, Hh:]

        return pl.pallas_call(
            kern_base,
            grid=(Nn // _BLK,),
            in_specs=in_specs,
            out_specs=pl.BlockSpec((2, _BLK, Hh), lambda i: (0, i, 0)),
            out_shape=jax.ShapeDtypeStruct((2, Nn, Hh), jnp.float32),
        )(*args)

    Kc = gc3W.shape[1]

    def kern_fused(h_ref, p_ref, gcw, gcb_, gw, gb_, lw, lb_, g3w,
                   o_ref, s_ref):
        hn = compute(h_ref, p_ref, gcw, gcb_, gw, gb_, lw, lb_)
        o_ref[0] = hn[:, :Hh]
        o_ref[1] = hn[:, Hh:]
        s_ref[...] = jnp.dot(hn, g3w[...], preferred_element_type=jnp.float32)

    return pl.pallas_call(
        kern_fused,
        grid=(Nn // _BLK,),
        in_specs=in_specs + [pl.BlockSpec((H, Kc), lambda i: (0, 0))],
        out_specs=[pl.BlockSpec((2, _BLK, Hh), lambda i: (0, i, 0)),
                   pl.BlockSpec((_BLK, Kc), lambda i: (i, 0))],
        out_shape=[jax.ShapeDtypeStruct((2, Nn, Hh), jnp.float32),
                   jax.ShapeDtypeStruct((Nn, Kc), jnp.float32)],
    )(*args, gc3W)


def _final_logsoftmax(p3, b3):
    _, Nn, Kc = p3.shape

    def kern(p_ref, b_ref, o_ref):
        v = p_ref[0] + p_ref[1] + b_ref[...]
        m = jnp.max(v, axis=1, keepdims=True)
        lse = jnp.log(jnp.sum(jnp.exp(v - m), axis=1, keepdims=True)) + m
        o_ref[...] = v - lse

    return pl.pallas_call(
        kern,
        grid=(Nn // _BLK,),
        in_specs=[pl.BlockSpec((NC, _BLK, Kc), lambda i: (0, i, 0)),
                  pl.BlockSpec((1, Kc), lambda i: (0, 0))],
        out_specs=pl.BlockSpec((_BLK, Kc), lambda i: (i, 0)),
        out_shape=jax.ShapeDtypeStruct((Nn, Kc), jnp.float32),
    )(p3, b3.reshape(1, -1))


def kernel(x, edge_index, W1, b1, hw1_gc_W, hw1_gc_b, hw1_gate_W, hw1_gate_b,
           hw1_lin_W, hw1_lin_b, hw2_gc_W, hw2_gc_b, hw2_gate_W, hw2_gate_b,
           hw2_lin_W, hw2_lin_b, gc3_W, gc3_b):
    E = edge_index.shape[1]
    src = edge_index[0].reshape(E // 128, 128)
    dst = edge_index[1].reshape(E // 128, 128)

    linear = _mm_act(x, W1, b1)

    p1 = _spmm(linear, src, dst, feat_split=True)
    conv1 = _highway(linear, p1, hw1_gc_W, hw1_gc_b, hw1_gate_W, hw1_gate_b,
                     hw1_lin_W, hw1_lin_b)

    p2 = _spmm(conv1, src, dst, feat_split=True)
    conv2, support3 = _highway(conv1, p2, hw2_gc_W, hw2_gc_b, hw2_gate_W,
                               hw2_gate_b, hw2_lin_W, hw2_lin_b, gc3W=gc3_W)

    p3 = _spmm(support3, src, dst, feat_split=False)
    return _final_logsoftmax(p3, gc3_b)
